# baseline (speedup 1.0000x reference)
import functools

import jax
import jax.numpy as jnp
from jax import lax
from jax.experimental import pallas as pl
from jax.experimental.pallas import tpu as pltpu
from jax.experimental.pallas import tpu_sc as plsc

N = 10000
D = 128
NC = 2
NS = 16
NW = NC * NS
NPAD = 10224
ROWS_PER_TILE = NPAD // NS
K = 64
DP = 144
BLK = 2000


def _make_sc_pass(d_cols, epw):
    nchunks = epw // K
    mesh = plsc.VectorSubcoreMesh(core_axis_name="c", subcore_axis_name="s")

    @functools.partial(
        pl.kernel,
        out_type=jax.ShapeDtypeStruct((NC * NPAD, d_cols), jnp.float32),
        mesh=mesh,
        scratch_types=[
            pltpu.VMEM((nchunks, K), jnp.int32),
            pltpu.VMEM((nchunks, K), jnp.int32),
            pltpu.VMEM((K, d_cols), jnp.float32),
            pltpu.VMEM((K, d_cols), jnp.float32),
            pltpu.VMEM_SHARED((NPAD, d_cols), jnp.float32),
            pltpu.SemaphoreType.DMA,
            pltpu.SemaphoreType.DMA,
        ],
        compiler_params=pltpu.CompilerParams(use_tc_tiling_on_sc=False),
    )
    def sc_pass(table_hbm, src_hbm, dst_hbm, out_hbm, sidx_v, didx_v,
                rbuf0, rbuf1, acc_sh, sem0, sem1):
        c = lax.axis_index("c")
        s = lax.axis_index("s")
        wid = s * NC + c

        nseg = d_cols // 16

        def zero_row(i, carry):
            for j in range(nseg):
                rbuf0[i, pl.ds(j * 16, 16)] = jnp.zeros((16,), jnp.float32)
            return carry

        lax.fori_loop(0, K, zero_row, 0)
        row0 = s * ROWS_PER_TILE
        for r in range(ROWS_PER_TILE // K):
            pltpu.sync_copy(rbuf0, acc_sh.at[pl.ds(row0 + r * K, K)])
        rem = ROWS_PER_TILE % K
        if rem:
            pltpu.sync_copy(
                rbuf0.at[pl.ds(0, rem)],
                acc_sh.at[pl.ds(row0 + (ROWS_PER_TILE // K) * K, rem)])

        pltpu.sync_copy(src_hbm.at[wid], sidx_v)
        pltpu.sync_copy(dst_hbm.at[wid], didx_v)
        plsc.subcore_barrier()

        def gather(t, buf, sem):
            pltpu.async_copy(table_hbm.at[sidx_v.at[t]], buf, sem)

        def gather_wait(t, buf, sem):
            pltpu.make_async_copy(
                table_hbm.at[sidx_v.at[t]], buf, sem).wait()

        def scatter(t, buf):
            pltpu.sync_copy(buf, acc_sh.at[didx_v.at[t]], add=True)

        npairs = nchunks // 2
        gather(0, rbuf0, sem0)

        def body(u, carry):
            t0 = 2 * u
            t1 = t0 + 1
            gather_wait(t0, rbuf0, sem0)
            gather(t1, rbuf1, sem1)
            scatter(t0, rbuf0)
            gather_wait(t1, rbuf1, sem1)

            @pl.when(u + 1 < npairs)
            def _():
                gather(t0 + 2, rbuf0, sem0)

            scatter(t1, rbuf1)
            return carry

        lax.fori_loop(0, npairs, body, 0)
        plsc.subcore_barrier()

        pltpu.sync_copy(
            acc_sh.at[pl.ds(s * ROWS_PER_TILE, ROWS_PER_TILE)],
            out_hbm.at[pl.ds(c * NPAD + s * ROWS_PER_TILE, ROWS_PER_TILE)])

    return sc_pass


def _tc_layer(acc, xin, wl_t, wr_t, b, relu_flag):
    def body(f_ref, acc_ref, x_ref, wl_ref, wr_ref, b_ref, o_ref):
        ssum = acc_ref[0] + acc_ref[1]
        cnt = ssum[:, 128:129]
        mean = ssum[:, :128] / jnp.maximum(cnt, 1.0)
        h = (jnp.dot(mean, wl_ref[...], preferred_element_type=jnp.float32)
             + jnp.dot(x_ref[:, :128], wr_ref[...],
                       preferred_element_type=jnp.float32)
             + b_ref[...])
        h = jnp.where(f_ref[0, 0] > 0.5, jnp.maximum(h, 0.0), h)
        o_ref[...] = jnp.concatenate(
            [h, jnp.ones((BLK, 1), jnp.float32),
             jnp.zeros((BLK, DP - D - 1), jnp.float32)], axis=1)

    return pl.pallas_call(
        body,
        grid=(N // BLK,),
        in_specs=[
            pl.BlockSpec((1, 1), lambda i: (0, 0)),
            pl.BlockSpec((2, BLK, DP), lambda i: (0, i, 0)),
            pl.BlockSpec((BLK, DP), lambda i: (i, 0)),
            pl.BlockSpec((D, D), lambda i: (0, 0)),
            pl.BlockSpec((D, D), lambda i: (0, 0)),
            pl.BlockSpec((1, D), lambda i: (0, 0)),
        ],
        out_specs=pl.BlockSpec((BLK, DP), lambda i: (i, 0)),
        out_shape=jax.ShapeDtypeStruct((N, DP), jnp.float32),
    )(relu_flag, acc, xin, wl_t, wr_t, b)


def kernel(x, adj_t, emb, Wl1, bl1, Wr1, Wl2, bl2, Wr2):
    src = adj_t[0].astype(jnp.int32)
    dst = adj_t[1].astype(jnp.int32)
    e = src.shape[0]
    epw = -(-e // (NW * 4 * K)) * 4 * K
    epad = NW * epw
    pad = epad - e
    src_p = jnp.concatenate(
        [src, jnp.zeros((pad,), jnp.int32)]).reshape(NW, epw // K, K)
    dst_p = jnp.concatenate(
        [dst, jnp.full((pad,), N, jnp.int32)]).reshape(NW, epw // K, K)

    table1 = jnp.concatenate(
        [emb, jnp.ones((N, 1), jnp.float32),
         jnp.zeros((N, DP - D - 1), jnp.float32)], axis=1)

    sc = _make_sc_pass(DP, epw)
    wl_s = jnp.stack([Wl1.T, Wl2.T])
    wr_s = jnp.stack([Wr1.T, Wr2.T])
    b_s = jnp.stack([bl1.reshape(1, D), bl2.reshape(1, D)])
    f_s = jnp.array([[[1.0]], [[0.0]]], jnp.float32)
    n_layers = lax.optimization_barrier(jnp.int32(2))

    def layer(state):
        i, table = state
        acc = sc(table, src_p, dst_p).reshape(NC, NPAD, DP)
        h = _tc_layer(acc, table, wl_s[i], wr_s[i], b_s[i], f_s[i])
        return (i + 1, h)

    _, h_final = lax.while_loop(
        lambda state: state[0] < n_layers, layer, (jnp.int32(0), table1))
    return h_final[:, :D]


# speedup vs baseline: 1.0452x; 1.0452x over previous
import functools

import jax
import jax.numpy as jnp
from jax import lax
from jax.experimental import pallas as pl
from jax.experimental.pallas import tpu as pltpu
from jax.experimental.pallas import tpu_sc as plsc

N = 10000
D = 128
NC = 2
NS = 16
NW = NC * NS
NPAD = 10224
ROWS_PER_TILE = NPAD // NS
K = 128
GC = 4
DP = 144
BLK = 2000


def _make_sc_pass(d_cols, epw):
    ngroups = epw // (GC * K)
    mesh = plsc.VectorSubcoreMesh(core_axis_name="c", subcore_axis_name="s")

    @functools.partial(
        pl.kernel,
        out_type=jax.ShapeDtypeStruct((NC * NPAD, d_cols), jnp.float32),
        mesh=mesh,
        scratch_types=[
            pltpu.VMEM((GC, K), jnp.int32),
            pltpu.VMEM((GC, K), jnp.int32),
            pltpu.VMEM((GC, K), jnp.int32),
            pltpu.VMEM((GC, K), jnp.int32),
            pltpu.VMEM((K, d_cols), jnp.float32),
            pltpu.VMEM((K, d_cols), jnp.float32),
            pltpu.VMEM_SHARED((NPAD, d_cols), jnp.float32),
            pltpu.SemaphoreType.DMA,
            pltpu.SemaphoreType.DMA,
            pltpu.SemaphoreType.DMA,
            pltpu.SemaphoreType.DMA,
        ],
        compiler_params=pltpu.CompilerParams(use_tc_tiling_on_sc=False),
    )
    def sc_pass(table_hbm, src_hbm, dst_hbm, out_hbm, sidx0, sidx1,
                didx0, didx1, rbuf0, rbuf1, acc_sh, semg0, semg1,
                semi0, semi1):
        c = lax.axis_index("c")
        s = lax.axis_index("s")
        wid = s * NC + c
        sidx = (sidx0, sidx1)
        didx = (didx0, didx1)
        rbuf = (rbuf0, rbuf1)
        semg = (semg0, semg1)
        semi = (semi0, semi1)

        nseg = d_cols // 16

        def zero_row(i, carry):
            for j in range(nseg):
                rbuf0[i, pl.ds(j * 16, 16)] = jnp.zeros((16,), jnp.float32)
            return carry

        lax.fori_loop(0, K, zero_row, 0)
        row0 = s * ROWS_PER_TILE
        for r in range(ROWS_PER_TILE // K):
            pltpu.sync_copy(rbuf0, acc_sh.at[pl.ds(row0 + r * K, K)])
        rem = ROWS_PER_TILE % K
        if rem:
            pltpu.sync_copy(
                rbuf0.at[pl.ds(0, rem)],
                acc_sh.at[pl.ds(row0 + (ROWS_PER_TILE // K) * K, rem)])

        plsc.subcore_barrier()

        def idx_issue(g, b):
            pltpu.async_copy(src_hbm.at[wid, g], sidx[b], semi[b])
            pltpu.async_copy(dst_hbm.at[wid, g], didx[b], semi[b])

        def idx_wait(g, b):
            pltpu.make_async_copy(src_hbm.at[wid, g], sidx[b], semi[b]).wait()
            pltpu.make_async_copy(dst_hbm.at[wid, g], didx[b], semi[b]).wait()

        def g_issue(b, j, r):
            pltpu.async_copy(table_hbm.at[sidx[b].at[j]], rbuf[r], semg[r])

        def g_wait(b, j, r):
            pltpu.make_async_copy(
                table_hbm.at[sidx[b].at[j]], rbuf[r], semg[r]).wait()

        def sc_add(b, j, r):
            pltpu.sync_copy(rbuf[r], acc_sh.at[didx[b].at[j]], add=True)

        def process_group(g, b):
            @pl.when(g + 1 < ngroups)
            def _():
                idx_issue(g + 1, 1 - b)

            for j in range(GC):
                r = j % 2
                g_wait(b, j, r)
                if j < GC - 1:
                    g_issue(b, j + 1, 1 - r)
                else:
                    @pl.when(g + 1 < ngroups)
                    def _():
                        idx_wait(g + 1, 1 - b)
                        g_issue(1 - b, 0, 1 - r)
                sc_add(b, j, r)

        idx_issue(0, 0)
        idx_wait(0, 0)
        g_issue(0, 0, 0)

        def body(gg, carry):
            process_group(2 * gg, 0)
            process_group(2 * gg + 1, 1)
            return carry

        lax.fori_loop(0, ngroups // 2, body, 0)
        plsc.subcore_barrier()

        pltpu.sync_copy(
            acc_sh.at[pl.ds(s * ROWS_PER_TILE, ROWS_PER_TILE)],
            out_hbm.at[pl.ds(c * NPAD + s * ROWS_PER_TILE, ROWS_PER_TILE)])

    return sc_pass


def _tc_layer(acc, xin, wl_t, wr_t, b, relu_flag):
    def body(f_ref, acc_ref, x_ref, wl_ref, wr_ref, b_ref, o_ref):
        ssum = acc_ref[0] + acc_ref[1]
        cnt = ssum[:, 128:129]
        mean = ssum[:, :128] / jnp.maximum(cnt, 1.0)
        h = (jnp.dot(mean, wl_ref[...], preferred_element_type=jnp.float32)
             + jnp.dot(x_ref[:, :128], wr_ref[...],
                       preferred_element_type=jnp.float32)
             + b_ref[...])
        h = jnp.where(f_ref[0, 0] > 0.5, jnp.maximum(h, 0.0), h)
        o_ref[...] = jnp.concatenate(
            [h, jnp.ones((BLK, 1), jnp.float32),
             jnp.zeros((BLK, DP - D - 1), jnp.float32)], axis=1)

    return pl.pallas_call(
        body,
        grid=(N // BLK,),
        in_specs=[
            pl.BlockSpec((1, 1), lambda i: (0, 0)),
            pl.BlockSpec((2, BLK, DP), lambda i: (0, i, 0)),
            pl.BlockSpec((BLK, DP), lambda i: (i, 0)),
            pl.BlockSpec((D, D), lambda i: (0, 0)),
            pl.BlockSpec((D, D), lambda i: (0, 0)),
            pl.BlockSpec((1, D), lambda i: (0, 0)),
        ],
        out_specs=pl.BlockSpec((BLK, DP), lambda i: (i, 0)),
        out_shape=jax.ShapeDtypeStruct((N, DP), jnp.float32),
    )(relu_flag, acc, xin, wl_t, wr_t, b)


def kernel(x, adj_t, emb, Wl1, bl1, Wr1, Wl2, bl2, Wr2):
    src = adj_t[0].astype(jnp.int32)
    dst = adj_t[1].astype(jnp.int32)
    e = src.shape[0]
    gsz = 2 * GC * K
    epw = -(-e // (NW * gsz)) * gsz
    epad = NW * epw
    pad = epad - e
    src_p = jnp.concatenate(
        [src, jnp.zeros((pad,), jnp.int32)]).reshape(NW, epw // (GC * K), GC, K)
    dst_p = jnp.concatenate(
        [dst, jnp.full((pad,), N, jnp.int32)]).reshape(NW, epw // (GC * K), GC, K)

    table1 = jnp.concatenate(
        [emb, jnp.ones((N, 1), jnp.float32),
         jnp.zeros((N, DP - D - 1), jnp.float32)], axis=1)

    sc = _make_sc_pass(DP, epw)
    wl_s = jnp.stack([Wl1.T, Wl2.T])
    wr_s = jnp.stack([Wr1.T, Wr2.T])
    b_s = jnp.stack([bl1.reshape(1, D), bl2.reshape(1, D)])
    f_s = jnp.array([[[1.0]], [[0.0]]], jnp.float32)
    n_layers = lax.optimization_barrier(jnp.int32(2))

    def layer(state):
        i, table = state
        acc = sc(table, src_p, dst_p).reshape(NC, NPAD, DP)
        h = _tc_layer(acc, table, wl_s[i], wr_s[i], b_s[i], f_s[i])
        return (i + 1, h)

    _, h_final = lax.while_loop(
        lambda state: state[0] < n_layers, layer, (jnp.int32(0), table1))
    return h_final[:, :D]


# speedup vs baseline: 2.0484x; 1.9597x over previous
import functools

import jax
import jax.numpy as jnp
from jax import lax
from jax.experimental import pallas as pl
from jax.experimental.pallas import tpu as pltpu
from jax.experimental.pallas import tpu_sc as plsc

N = 10000
D = 128
NC = 2
NS = 16
NPAD = 10224
ROWS_PER_TILE = NPAD // NS
TAB_ROWS_PER_TILE = N // NS
DH = 80
FB = 48
K = 128
GC = 4
BLK = 2000


def _make_sc_pass(epw):
    ngroups = epw // (GC * K)
    mesh = plsc.VectorSubcoreMesh(core_axis_name="c", subcore_axis_name="s")

    @functools.partial(
        pl.kernel,
        out_type=jax.ShapeDtypeStruct((NC * NPAD, DH), jnp.float32),
        mesh=mesh,
        scratch_types=[
            pltpu.VMEM((GC, K), jnp.int32),
            pltpu.VMEM((GC, K), jnp.int32),
            pltpu.VMEM((GC, K), jnp.int32),
            pltpu.VMEM((GC, K), jnp.int32),
            pltpu.VMEM((K, DH), jnp.float32),
            pltpu.VMEM((K, DH), jnp.float32),
            pltpu.VMEM_SHARED((N, DH), jnp.float32),
            pltpu.VMEM_SHARED((NPAD, DH), jnp.float32),
            pltpu.SemaphoreType.DMA,
            pltpu.SemaphoreType.DMA,
            pltpu.SemaphoreType.DMA,
            pltpu.SemaphoreType.DMA,
        ],
        compiler_params=pltpu.CompilerParams(use_tc_tiling_on_sc=False),
    )
    def sc_pass(tabs_hbm, src_hbm, dst_hbm, out_hbm, sidx0, sidx1,
                didx0, didx1, rbuf0, rbuf1, tab_sh, acc_sh,
                semg0, semg1, semi0, semi1):
        c = lax.axis_index("c")
        s = lax.axis_index("s")
        sidx = (sidx0, sidx1)
        didx = (didx0, didx1)
        rbuf = (rbuf0, rbuf1)
        semg = (semg0, semg1)
        semi = (semi0, semi1)

        trow = s * TAB_ROWS_PER_TILE
        pltpu.async_copy(
            tabs_hbm.at[c, pl.ds(trow, TAB_ROWS_PER_TILE)],
            tab_sh.at[pl.ds(trow, TAB_ROWS_PER_TILE)], semg0)

        def zero_row(i, carry):
            for j in range(DH // 16):
                rbuf0[i, pl.ds(j * 16, 16)] = jnp.zeros((16,), jnp.float32)
            return carry

        lax.fori_loop(0, K, zero_row, 0)
        pltpu.make_async_copy(
            tabs_hbm.at[c, pl.ds(trow, TAB_ROWS_PER_TILE)],
            tab_sh.at[pl.ds(trow, TAB_ROWS_PER_TILE)], semg0).wait()

        row0 = s * ROWS_PER_TILE
        for r in range(ROWS_PER_TILE // K):
            pltpu.sync_copy(rbuf0, acc_sh.at[pl.ds(row0 + r * K, K)])
        rem = ROWS_PER_TILE % K
        if rem:
            pltpu.sync_copy(
                rbuf0.at[pl.ds(0, rem)],
                acc_sh.at[pl.ds(row0 + (ROWS_PER_TILE // K) * K, rem)])
        plsc.subcore_barrier()

        def idx_issue(g, b):
            pltpu.async_copy(src_hbm.at[s, g], sidx[b], semi[b])
            pltpu.async_copy(dst_hbm.at[s, g], didx[b], semi[b])

        def idx_wait(g, b):
            pltpu.make_async_copy(src_hbm.at[s, g], sidx[b], semi[b]).wait()
            pltpu.make_async_copy(dst_hbm.at[s, g], didx[b], semi[b]).wait()

        def g_issue(b, j, r):
            pltpu.async_copy(tab_sh.at[sidx[b].at[j]], rbuf[r], semg[r])

        def g_wait(b, j, r):
            pltpu.make_async_copy(
                tab_sh.at[sidx[b].at[j]], rbuf[r], semg[r]).wait()

        def sc_add(b, j, r):
            pltpu.sync_copy(rbuf[r], acc_sh.at[didx[b].at[j]], add=True)

        def process_group(g, b):
            @pl.when(g + 1 < ngroups)
            def _():
                idx_issue(g + 1, 1 - b)

            for j in range(GC):
                r = j % 2
                if j < GC - 1:
                    g_issue(b, j + 1, 1 - r)
                else:
                    @pl.when(g + 1 < ngroups)
                    def _():
                        idx_wait(g + 1, 1 - b)
                        g_issue(1 - b, 0, 1 - r)
                g_wait(b, j, r)
                sc_add(b, j, r)

        idx_issue(0, 0)
        idx_wait(0, 0)
        g_issue(0, 0, 0)

        def body(gg, carry):
            process_group(2 * gg, 0)
            process_group(2 * gg + 1, 1)
            return carry

        lax.fori_loop(0, ngroups // 2, body, 0)
        plsc.subcore_barrier()

        pltpu.sync_copy(
            acc_sh.at[pl.ds(row0, ROWS_PER_TILE)],
            out_hbm.at[pl.ds(c * NPAD + row0, ROWS_PER_TILE)])

    return sc_pass


def _tc_layer(acc, ta, tb, wl_t, wr_t, b, relu_flag):
    def body(f_ref, acc_ref, ta_ref, tb_ref, wl_ref, wr_ref, b_ref,
             oa_ref, ob_ref):
        a0 = acc_ref[0]
        a1 = acc_ref[1]
        cnt = jnp.maximum(a1[:, FB:FB + 1], 1.0)
        mean = jnp.concatenate([a0, a1[:, :FB]], axis=1) / cnt
        x = jnp.concatenate([ta_ref[...], tb_ref[:, :FB]], axis=1)
        h = (jnp.dot(mean, wl_ref[...], preferred_element_type=jnp.float32)
             + jnp.dot(x, wr_ref[...], preferred_element_type=jnp.float32)
             + b_ref[...])
        h = jnp.where(f_ref[0, 0] > 0.5, jnp.maximum(h, 0.0), h)
        oa_ref[...] = h[:, :DH]
        ob_ref[...] = jnp.concatenate(
            [h[:, DH:], jnp.ones((BLK, 1), jnp.float32),
             jnp.zeros((BLK, DH - FB - 1), jnp.float32)], axis=1)

    return pl.pallas_call(
        body,
        grid=(N // BLK,),
        in_specs=[
            pl.BlockSpec((1, 1), lambda i: (0, 0)),
            pl.BlockSpec((2, BLK, DH), lambda i: (0, i, 0)),
            pl.BlockSpec((BLK, DH), lambda i: (i, 0)),
            pl.BlockSpec((BLK, DH), lambda i: (i, 0)),
            pl.BlockSpec((D, D), lambda i: (0, 0)),
            pl.BlockSpec((D, D), lambda i: (0, 0)),
            pl.BlockSpec((1, D), lambda i: (0, 0)),
        ],
        out_specs=[
            pl.BlockSpec((BLK, DH), lambda i: (i, 0)),
            pl.BlockSpec((BLK, DH), lambda i: (i, 0)),
        ],
        out_shape=[
            jax.ShapeDtypeStruct((N, DH), jnp.float32),
            jax.ShapeDtypeStruct((N, DH), jnp.float32),
        ],
    )(relu_flag, acc, ta, tb, wl_t, wr_t, b)


def kernel(x, adj_t, emb, Wl1, bl1, Wr1, Wl2, bl2, Wr2):
    src = adj_t[0].astype(jnp.int32)
    dst = adj_t[1].astype(jnp.int32)
    e = src.shape[0]
    gsz = 2 * GC * K
    epw = -(-e // (NS * gsz)) * gsz
    epad = NS * epw
    pad = epad - e
    src_p = jnp.concatenate(
        [src, jnp.zeros((pad,), jnp.int32)]).reshape(NS, epw // (GC * K), GC, K)
    dst_p = jnp.concatenate(
        [dst, jnp.full((pad,), N, jnp.int32)]).reshape(NS, epw // (GC * K), GC, K)

    ta1 = emb[:, :DH]
    tb1 = jnp.concatenate(
        [emb[:, DH:], jnp.ones((N, 1), jnp.float32),
         jnp.zeros((N, DH - FB - 1), jnp.float32)], axis=1)

    sc = _make_sc_pass(epw)
    wl_s = jnp.stack([Wl1.T, Wl2.T])
    wr_s = jnp.stack([Wr1.T, Wr2.T])
    b_s = jnp.stack([bl1.reshape(1, D), bl2.reshape(1, D)])
    f_s = jnp.array([[[1.0]], [[0.0]]], jnp.float32)
    n_layers = lax.optimization_barrier(jnp.int32(2))

    def layer(state):
        i, ta, tb = state
        tabs = jnp.stack([ta, tb])
        acc = sc(tabs, src_p, dst_p).reshape(NC, NPAD, DH)
        ha, hb = _tc_layer(acc, ta, tb, wl_s[i], wr_s[i], b_s[i], f_s[i])
        return (i + 1, ha, hb)

    _, ha, hb = lax.while_loop(
        lambda state: state[0] < n_layers, layer,
        (jnp.int32(0), ta1, tb1))
    return jnp.concatenate([ha, hb[:, :FB]], axis=1)


# speedup vs baseline: 2.0570x; 1.0042x over previous
import functools

import jax
import jax.numpy as jnp
from jax import lax
from jax.experimental import pallas as pl
from jax.experimental.pallas import tpu as pltpu
from jax.experimental.pallas import tpu_sc as plsc

N = 10000
D = 128
NC = 2
NS = 16
NPAD = 10224
ROWS_PER_TILE = NPAD // NS
TAB_ROWS_PER_TILE = N // NS
DH = 80
FB = 48
K = 128
GC = 4
BLK = 2000


def _make_sc_pass(epw):
    ngroups = epw // (GC * K)
    mesh = plsc.VectorSubcoreMesh(core_axis_name="c", subcore_axis_name="s")

    @functools.partial(
        pl.kernel,
        out_type=jax.ShapeDtypeStruct((NC * NPAD, DH), jnp.float32),
        mesh=mesh,
        scratch_types=[
            pltpu.VMEM((GC, K), jnp.int32),
            pltpu.VMEM((GC, K), jnp.int32),
            pltpu.VMEM((GC, K), jnp.int32),
            pltpu.VMEM((GC, K), jnp.int32),
            pltpu.VMEM((K, DH), jnp.float32),
            pltpu.VMEM((K, DH), jnp.float32),
            pltpu.VMEM_SHARED((N, DH), jnp.float32),
            pltpu.VMEM_SHARED((NPAD, DH), jnp.float32),
            pltpu.SemaphoreType.DMA,
            pltpu.SemaphoreType.DMA,
            pltpu.SemaphoreType.DMA,
            pltpu.SemaphoreType.DMA,
            pltpu.SemaphoreType.DMA,
            pltpu.SemaphoreType.DMA,
        ],
        compiler_params=pltpu.CompilerParams(use_tc_tiling_on_sc=False),
    )
    def sc_pass(tabs_hbm, src_hbm, dst_hbm, out_hbm, sidx0, sidx1,
                didx0, didx1, rbuf0, rbuf1, tab_sh, acc_sh,
                semg0, semg1, semi0, semi1, semsc0, semsc1):
        c = lax.axis_index("c")
        s = lax.axis_index("s")
        sidx = (sidx0, sidx1)
        didx = (didx0, didx1)
        rbuf = (rbuf0, rbuf1)
        semg = (semg0, semg1)
        semi = (semi0, semi1)
        semsc = (semsc0, semsc1)

        trow = s * TAB_ROWS_PER_TILE
        pltpu.async_copy(
            tabs_hbm.at[c, pl.ds(trow, TAB_ROWS_PER_TILE)],
            tab_sh.at[pl.ds(trow, TAB_ROWS_PER_TILE)], semg0)

        def zero_row(i, carry):
            for j in range(DH // 16):
                rbuf0[i, pl.ds(j * 16, 16)] = jnp.zeros((16,), jnp.float32)
            return carry

        lax.fori_loop(0, K, zero_row, 0)
        pltpu.make_async_copy(
            tabs_hbm.at[c, pl.ds(trow, TAB_ROWS_PER_TILE)],
            tab_sh.at[pl.ds(trow, TAB_ROWS_PER_TILE)], semg0).wait()

        row0 = s * ROWS_PER_TILE
        for r in range(ROWS_PER_TILE // K):
            pltpu.sync_copy(rbuf0, acc_sh.at[pl.ds(row0 + r * K, K)])
        rem = ROWS_PER_TILE % K
        if rem:
            pltpu.sync_copy(
                rbuf0.at[pl.ds(0, rem)],
                acc_sh.at[pl.ds(row0 + (ROWS_PER_TILE // K) * K, rem)])
        plsc.subcore_barrier()

        def idx_issue(g, b):
            pltpu.async_copy(src_hbm.at[s, g], sidx[b], semi[b])
            pltpu.async_copy(dst_hbm.at[s, g], didx[b], semi[b])

        def idx_wait(g, b):
            pltpu.make_async_copy(src_hbm.at[s, g], sidx[b], semi[b]).wait()
            pltpu.make_async_copy(dst_hbm.at[s, g], didx[b], semi[b]).wait()

        def g_issue(b, j, r):
            pltpu.async_copy(tab_sh.at[sidx[b].at[j]], rbuf[r], semg[r])

        def g_wait(b, j, r):
            pltpu.make_async_copy(
                tab_sh.at[sidx[b].at[j]], rbuf[r], semg[r]).wait()

        def process_group(g, b):
            @pl.when(g + 1 < ngroups)
            def _():
                idx_issue(g + 1, 1 - b)

            for j in range(GC):
                r = j % 2
                g_wait(b, j, r)
                hsc = pltpu.async_copy(
                    rbuf[r], acc_sh.at[didx[b].at[j]], semsc0, add=True)
                if j < GC - 1:
                    g_issue(b, j + 1, 1 - r)
                else:
                    @pl.when(g + 1 < ngroups)
                    def _():
                        idx_wait(g + 1, 1 - b)
                        g_issue(1 - b, 0, 1 - r)
                hsc.wait()

        idx_issue(0, 0)
        idx_wait(0, 0)
        g_issue(0, 0, 0)

        def body(gg, carry):
            process_group(2 * gg, 0)
            process_group(2 * gg + 1, 1)
            return carry

        lax.fori_loop(0, ngroups // 2, body, 0)
        plsc.subcore_barrier()

        pltpu.sync_copy(
            acc_sh.at[pl.ds(row0, ROWS_PER_TILE)],
            out_hbm.at[pl.ds(c * NPAD + row0, ROWS_PER_TILE)])

    return sc_pass


def _tc_layer(acc, ta, wl_t, wr_t, b, relu_flag):
    def body(f_ref, acc_ref, t_ref, wl_ref, wr_ref, b_ref, o_ref):
        a0 = acc_ref[0]
        a1 = acc_ref[1]
        cnt = jnp.maximum(a1[:, FB:FB + 1], 1.0)
        mean = jnp.concatenate([a0, a1[:, :FB]], axis=1) / cnt
        x = jnp.concatenate([t_ref[0], t_ref[1, :, :FB]], axis=1)
        h = (jnp.dot(mean, wl_ref[...], preferred_element_type=jnp.float32)
             + jnp.dot(x, wr_ref[...], preferred_element_type=jnp.float32)
             + b_ref[...])
        h = jnp.where(f_ref[0, 0] > 0.5, jnp.maximum(h, 0.0), h)
        o_ref[0] = h[:, :DH]
        o_ref[1] = jnp.concatenate(
            [h[:, DH:], jnp.ones((BLK, 1), jnp.float32),
             jnp.zeros((BLK, DH - FB - 1), jnp.float32)], axis=1)

    return pl.pallas_call(
        body,
        grid=(N // BLK,),
        in_specs=[
            pl.BlockSpec((1, 1), lambda i: (0, 0)),
            pl.BlockSpec((2, BLK, DH), lambda i: (0, i, 0)),
            pl.BlockSpec((2, BLK, DH), lambda i: (0, i, 0)),
            pl.BlockSpec((D, D), lambda i: (0, 0)),
            pl.BlockSpec((D, D), lambda i: (0, 0)),
            pl.BlockSpec((1, D), lambda i: (0, 0)),
        ],
        out_specs=pl.BlockSpec((2, BLK, DH), lambda i: (0, i, 0)),
        out_shape=jax.ShapeDtypeStruct((2, N, DH), jnp.float32),
    )(relu_flag, acc, ta, wl_t, wr_t, b)


def kernel(x, adj_t, emb, Wl1, bl1, Wr1, Wl2, bl2, Wr2):
    src = adj_t[0].astype(jnp.int32)
    dst = adj_t[1].astype(jnp.int32)
    e = src.shape[0]
    gsz = 2 * GC * K
    epw = -(-e // (NS * gsz)) * gsz
    epad = NS * epw
    pad = epad - e
    src_p = jnp.concatenate(
        [src, jnp.zeros((pad,), jnp.int32)]).reshape(NS, epw // (GC * K), GC, K)
    dst_p = jnp.concatenate(
        [dst, jnp.full((pad,), N, jnp.int32)]).reshape(NS, epw // (GC * K), GC, K)

    tabs1 = jnp.stack([
        emb[:, :DH],
        jnp.concatenate(
            [emb[:, DH:], jnp.ones((N, 1), jnp.float32),
             jnp.zeros((N, DH - FB - 1), jnp.float32)], axis=1),
    ])

    sc = _make_sc_pass(epw)
    wl_s = jnp.stack([Wl1.T, Wl2.T])
    wr_s = jnp.stack([Wr1.T, Wr2.T])
    b_s = jnp.stack([bl1.reshape(1, D), bl2.reshape(1, D)])
    f_s = jnp.array([[[1.0]], [[0.0]]], jnp.float32)
    n_layers = lax.optimization_barrier(jnp.int32(2))

    def layer(state):
        i, tabs = state
        acc = sc(tabs, src_p, dst_p).reshape(NC, NPAD, DH)
        h = _tc_layer(acc, tabs, wl_s[i], wr_s[i], b_s[i], f_s[i])
        return (i + 1, h)

    _, h = lax.while_loop(
        lambda state: state[0] < n_layers, layer, (jnp.int32(0), tabs1))
    return jnp.concatenate([h[0], h[1, :, :FB]], axis=1)
